# baseline (speedup 1.0000x reference)
import functools

import jax
import jax.numpy as jnp
from jax import lax
from jax.experimental import pallas as pl
from jax.experimental.pallas import tpu as pltpu
from jax.experimental.pallas import tpu_sc as plsc

DIM = 128
PK = DIM // 2
MARGIN = 4.0
BATCH = 4096
NROWS = 1000
NC = 2
NS = 16
NW = NC * NS
L = 16
TRIPLES_PER_W = BATCH // NW
GROUPS = TRIPLES_PER_W // L
IDS = 1024


def _rsqrt(x):
    x = jnp.maximum(x, 1e-30)
    i = plsc.bitcast(x, jnp.int32)
    i = 0x5F3759DF - (i >> 1)
    y = plsc.bitcast(i, jnp.float32)
    for _ in range(3):
        y = y * (1.5 - 0.5 * x * y * y)
    return y


def _unpack(w):
    lo = plsc.bitcast(w << 16, jnp.float32)
    hi = plsc.bitcast(w & jnp.int32(-65536), jnp.float32)
    return lo, hi


def _cross_dots(ebuf, rbuf, qbuf, lanes, ho, to, ro):
    zeros = jnp.zeros((L,), jnp.float32)
    hrow = lanes + ho
    trow = lanes + to
    rrow = lanes + ro

    def body(d, c):
        dcol = (jnp.broadcast_to(d, (L,)).astype(jnp.int32) + lanes) & (PK - 1)
        h0, h1 = _unpack(plsc.load_gather(ebuf, [hrow, dcol]))
        t0, t1 = _unpack(plsc.load_gather(ebuf, [trow, dcol]))
        r0, r1 = _unpack(plsc.load_gather(rbuf, [rrow, dcol]))
        q0, q1 = _unpack(plsc.load_gather(qbuf, [rrow, dcol]))
        return (
            c[0] + h0 * r0 + h1 * r1,
            c[1] + h0 * t0 + h1 * t1,
            c[2] + r0 * t0 + r1 * t1,
            c[3] + h0 * q0 + h1 * q1,
            c[4] + t0 * q0 + t1 * q1,
        )

    return lax.fori_loop(0, PK, body, (zeros,) * 5)


def _make_sc_kernel():
    mesh = plsc.VectorSubcoreMesh(core_axis_name="c", subcore_axis_name="s")

    @functools.partial(
        pl.kernel,
        mesh=mesh,
        compiler_params=pltpu.CompilerParams(needs_layout_passes=False, use_tc_tiling_on_sc=False),
        out_type=jax.ShapeDtypeStruct((NW, L), jnp.float32),
        scratch_types=(
            [pltpu.VMEM((GROUPS, 4 * L), jnp.int32),
             pltpu.VMEM((GROUPS, 2 * L), jnp.int32),
             pltpu.VMEM((4 * L, PK), jnp.int32),
             pltpu.VMEM((4 * L, PK), jnp.int32),
             pltpu.VMEM((2 * L, PK), jnp.int32),
             pltpu.VMEM((2 * L, PK), jnp.int32),
             pltpu.VMEM((2 * L, PK), jnp.int32),
             pltpu.VMEM((2 * L, PK), jnp.int32)]
            + [pltpu.VMEM((IDS,), jnp.float32) for _ in range(2)]
            + [pltpu.VMEM((NROWS,), jnp.float32) for _ in range(3)]
            + [pltpu.VMEM((L,), jnp.float32),
               pltpu.SemaphoreType.DMA,
               pltpu.SemaphoreType.DMA,
               pltpu.SemaphoreType.DMA]
        ),
    )
    def sc_kernel(ee_hbm, eet_hbm, rr_hbm, rrt_hbm, qq_hbm, epk, rpk, qpk,
                  eidx_hbm, ridx_hbm, out_hbm, eidx_v, ridx_v,
                  ebA, ebB, rbA, rbB, qbA, qbB,
                  pre0, pre1, pre2, pre3, pre4, acc_v,
                  semA, semB, semP):
        cid = lax.axis_index("c")
        sid = lax.axis_index("s")
        wid = sid * NC + cid
        lanes = lax.iota(jnp.int32, L)

        pltpu.sync_copy(eidx_hbm.at[wid], eidx_v)
        pltpu.sync_copy(ridx_hbm.at[wid], ridx_v)

        sets = ((ebA, rbA, qbA, semA), (ebB, rbB, qbB, semB))

        def fire(g, s):
            eb, rb, qb, sem = s
            pltpu.async_copy(epk.at[eidx_v.at[g]], eb, sem)
            pltpu.async_copy(rpk.at[ridx_v.at[g]], rb, sem)
            pltpu.async_copy(qpk.at[ridx_v.at[g]], qb, sem)

        def drain(s):
            eb, rb, qb, sem = s
            pltpu.make_async_copy(epk.at[eidx_v.at[0]], eb, sem).wait()
            pltpu.make_async_copy(rpk.at[ridx_v.at[0]], rb, sem).wait()
            pltpu.make_async_copy(qpk.at[ridx_v.at[0]], qb, sem).wait()

        fire(0, sets[0])
        fire(1, sets[1])

        c1 = pltpu.async_copy(ee_hbm, pre0, semP)
        c2 = pltpu.async_copy(eet_hbm, pre1, semP)
        c3 = pltpu.async_copy(rr_hbm, pre2, semP)
        c4 = pltpu.async_copy(rrt_hbm, pre3, semP)
        c5 = pltpu.async_copy(qq_hbm, pre4, semP)
        for c in (c1, c2, c3, c4, c5):
            c.wait()

        def distance(s, g, ho, to, ro):
            eb, rb, qb, _ = s
            h_ids = eidx_v[g, pl.ds(ho * L, L)]
            t_ids = eidx_v[g, pl.ds(to * L, L)]
            r_ids = ridx_v[g, pl.ds(ro * L, L)]
            hh = plsc.load_gather(pre0, [h_ids])
            tt2 = plsc.load_gather(pre0, [t_ids])
            sh = plsc.load_gather(pre1, [h_ids])
            st = plsc.load_gather(pre1, [t_ids])
            rr = plsc.load_gather(pre2, [r_ids])
            rrt = plsc.load_gather(pre3, [r_ids])
            rtrt = plsc.load_gather(pre4, [r_ids])
            hr, ht_d, rt_d, hrt, trt = _cross_dots(
                eb, rb, qb, lanes, ho * L, to * L, ro * L)
            nh2 = hh + 2.0 * sh * hrt + sh * sh * rtrt
            nt2 = tt2 + 2.0 * st * trt + st * st * rtrt
            hp_r = hr + sh * rrt
            hp_tp = ht_d + st * hrt + sh * trt + sh * st * rtrt
            r_tp = rt_d + st * rrt
            inh = _rsqrt(nh2)
            int_ = _rsqrt(nt2)
            inr = _rsqrt(rr)
            d2 = 3.0 + 2.0 * (hp_r * inh * inr - hp_tp * inh * int_
                              - r_tp * inr * int_)
            d2 = jnp.maximum(d2, 0.0)
            return d2 * _rsqrt(d2)

        def compute(s, g, acc):
            pos = distance(s, g, 0, 1, 0)
            neg = distance(s, g, 2, 3, 1)
            return acc + jnp.maximum(pos - neg + MARGIN, 0.0)

        def pair(gg, acc):
            drain(sets[0])
            acc = compute(sets[0], 2 * gg, acc)

            @pl.when(gg < GROUPS // 2 - 1)
            def _():
                fire(2 * gg + 2, sets[0])

            drain(sets[1])
            acc = compute(sets[1], 2 * gg + 1, acc)

            @pl.when(gg < GROUPS // 2 - 1)
            def _():
                fire(2 * gg + 3, sets[1])

            return acc

        acc = lax.fori_loop(0, GROUPS // 2, pair, jnp.zeros((L,), jnp.float32))
        acc_v[...] = acc
        pltpu.sync_copy(acc_v, out_hbm.at[wid])

    return sc_kernel


_SC_KERNEL = _make_sc_kernel()


def _prep_body(ent_ref, enttr_ref, rel_ref, reltr_ref,
               ee_ref, eet_ref, rr_ref, rrt_ref, qq_ref):
    e = ent_ref[...]
    et = enttr_ref[...]
    ee_ref[...] = jnp.sum(e * e, axis=1, keepdims=True)
    eet_ref[...] = jnp.sum(e * et, axis=1, keepdims=True)
    r = rel_ref[...]
    rt = reltr_ref[...]
    rr_ref[...] = jnp.sum(r * r, axis=1, keepdims=True)
    rrt_ref[...] = jnp.sum(r * rt, axis=1, keepdims=True)
    qq_ref[...] = jnp.sum(rt * rt, axis=1, keepdims=True)


_PREP = pl.pallas_call(
    _prep_body,
    grid=(1,),
    in_specs=[
        pl.BlockSpec((IDS, DIM), lambda i: (0, 0)),
        pl.BlockSpec((IDS, DIM), lambda i: (0, 0)),
        pl.BlockSpec((NROWS, DIM), lambda i: (0, 0)),
        pl.BlockSpec((NROWS, DIM), lambda i: (0, 0)),
    ],
    out_specs=[
        pl.BlockSpec((IDS, 1), lambda i: (0, 0)),
        pl.BlockSpec((IDS, 1), lambda i: (0, 0)),
        pl.BlockSpec((NROWS, 1), lambda i: (0, 0)),
        pl.BlockSpec((NROWS, 1), lambda i: (0, 0)),
        pl.BlockSpec((NROWS, 1), lambda i: (0, 0)),
    ],
    out_shape=[
        jax.ShapeDtypeStruct((IDS, 1), jnp.float32),
        jax.ShapeDtypeStruct((IDS, 1), jnp.float32),
        jax.ShapeDtypeStruct((NROWS, 1), jnp.float32),
        jax.ShapeDtypeStruct((NROWS, 1), jnp.float32),
        jax.ShapeDtypeStruct((NROWS, 1), jnp.float32),
    ],
)


def _pack_bf16(table):
    b = table.astype(jnp.bfloat16)
    lo = lax.bitcast_convert_type(b[:, :PK], jnp.uint16).astype(jnp.uint32)
    hi = lax.bitcast_convert_type(b[:, PK:], jnp.uint16).astype(jnp.uint32)
    return lax.bitcast_convert_type(lo | (hi << 16), jnp.int32)


@jax.jit
def kernel(current_triples, corrupted_triples, ent_embedding, rel_embedding,
           ent_transfer, rel_transfer):
    cur = current_triples.astype(jnp.int32)
    cor = corrupted_triples.astype(jnp.int32)

    def wg(col_arrays):
        parts = [a.reshape(NW, GROUPS, L) for a in col_arrays]
        return jnp.stack(parts, axis=2).reshape(NW, GROUPS, len(parts) * L)

    eidx = wg([cur[:, 0], cur[:, 2], cor[:, 0], cor[:, 2]])
    ridx = wg([cur[:, 1], cor[:, 1]])
    epk = _pack_bf16(ent_embedding[:NROWS])
    rpk = _pack_bf16(rel_embedding)
    qpk = _pack_bf16(rel_transfer)
    ee, eet, rr, rrt, qq = _PREP(ent_embedding, ent_transfer,
                                 rel_embedding, rel_transfer)
    partials = _SC_KERNEL(ee.reshape(IDS), eet.reshape(IDS),
                          rr.reshape(NROWS), rrt.reshape(NROWS),
                          qq.reshape(NROWS), epk, rpk, qpk, eidx, ridx)
    return jnp.sum(partials) / BATCH


# speedup vs baseline: 1.6502x; 1.6502x over previous
import functools

import jax
import jax.numpy as jnp
from jax import lax
from jax.experimental import pallas as pl
from jax.experimental.pallas import tpu as pltpu
from jax.experimental.pallas import tpu_sc as plsc

DIM = 128
PK = DIM // 2
MARGIN = 4.0
BATCH = 4096
NROWS = 1000
NC = 2
NS = 16
NW = NC * NS
L = 16
TRIPLES_PER_W = BATCH // NW
GROUPS = TRIPLES_PER_W // L
IDS = 1024


def _rsqrt(x):
    x = jnp.maximum(x, 1e-30)
    i = plsc.bitcast(x, jnp.int32)
    i = 0x5F3759DF - (i >> 1)
    y = plsc.bitcast(i, jnp.float32)
    for _ in range(3):
        y = y * (1.5 - 0.5 * x * y * y)
    return y


def _unpack(w):
    lo = plsc.bitcast(w << 16, jnp.float32)
    hi = plsc.bitcast(w & jnp.int32(-65536), jnp.float32)
    return lo, hi


def _cross_dots(ebuf, wbuf, lanes, ho, to, ro):
    zeros = jnp.zeros((L,), jnp.float32)
    hrow = lanes + ho
    trow = lanes + to
    rrow = lanes + ro

    def body(d, c):
        dcol = (jnp.broadcast_to(d, (L,)).astype(jnp.int32) + lanes) & (DIM - 1)
        h = plsc.load_gather(ebuf, [hrow, dcol])
        t = plsc.load_gather(ebuf, [trow, dcol])
        r, q = _unpack(plsc.load_gather(wbuf, [rrow, dcol]))
        return (
            c[0] + h * r,
            c[1] + h * t,
            c[2] + r * t,
            c[3] + h * q,
            c[4] + t * q,
        )

    return lax.fori_loop(0, DIM, body, (zeros,) * 5)


def _make_sc_kernel():
    mesh = plsc.VectorSubcoreMesh(core_axis_name="c", subcore_axis_name="s")

    @functools.partial(
        pl.kernel,
        mesh=mesh,
        compiler_params=pltpu.CompilerParams(needs_layout_passes=False, use_tc_tiling_on_sc=False),
        out_type=jax.ShapeDtypeStruct((NW, L), jnp.float32),
        scratch_types=(
            [pltpu.VMEM((GROUPS, 4 * L), jnp.int32),
             pltpu.VMEM((GROUPS, 2 * L), jnp.int32),
             pltpu.VMEM((4 * L, DIM), jnp.float32),
             pltpu.VMEM((4 * L, DIM), jnp.float32),
             pltpu.VMEM((2 * L, DIM), jnp.int32),
             pltpu.VMEM((2 * L, DIM), jnp.int32),
             pltpu.VMEM((4 * L, DIM), jnp.float32),
             pltpu.VMEM((4 * L, DIM), jnp.float32)]
            + [pltpu.VMEM((4 * L,), jnp.float32) for _ in range(5)]
            + [pltpu.VMEM((IDS,), jnp.float32) for _ in range(5)]
            + [pltpu.VMEM_SHARED((IDS,), jnp.float32) for _ in range(5)]
            + [pltpu.VMEM((L,), jnp.float32),
               pltpu.SemaphoreType.DMA,
               pltpu.SemaphoreType.DMA,
               pltpu.SemaphoreType.DMA]
        ),
    )
    def sc_kernel(ent_emb, ent_tr, rel_emb, rel_tr, wrel,
                  eidx_hbm, ridx_hbm, out_hbm, eidx_v, ridx_v,
                  ebA, ebB, wrA, wrB, ta, tb,
                  loc0, loc1, loc2, loc3, loc4,
                  pre0, pre1, pre2, pre3, pre4,
                  sh0, sh1, sh2, sh3, sh4, acc_v,
                  semA, semB, semP):
        cid = lax.axis_index("c")
        sid = lax.axis_index("s")
        wid = sid * NC + cid
        lanes = lax.iota(jnp.int32, L)

        pltpu.sync_copy(eidx_hbm.at[wid], eidx_v)
        pltpu.sync_copy(ridx_hbm.at[wid], ridx_v)

        sets = ((ebA, wrA, semA), (ebB, wrB, semB))

        def fire(g, s):
            eb, wr, sem = s
            pltpu.async_copy(ent_emb.at[eidx_v.at[g]], eb, sem)
            pltpu.async_copy(wrel.at[ridx_v.at[g]], wr, sem)

        def drain(s):
            eb, wr, sem = s
            pltpu.make_async_copy(ent_emb.at[eidx_v.at[0]], eb, sem).wait()
            pltpu.make_async_copy(wrel.at[ridx_v.at[0]], wr, sem).wait()

        fire(0, sets[0])
        fire(1, sets[1])

        base_e = pl.multiple_of(sid * (4 * L), 4 * L)
        base_r = pl.multiple_of(jnp.minimum(base_e, NROWS - 4 * L), 8)
        c1 = pltpu.async_copy(ent_emb.at[pl.ds(base_e, 4 * L)], ta, semP)
        c2 = pltpu.async_copy(ent_tr.at[pl.ds(base_e, 4 * L)], tb, semP)
        c1.wait()
        c2.wait()

        def diag_dots(sub, nacc):
            rows = lanes + sub * L

            def body(d, c):
                dcol = (jnp.broadcast_to(d, (L,)).astype(jnp.int32)
                        + lanes) & (DIM - 1)
                a = plsc.load_gather(ta, [rows, dcol])
                b = plsc.load_gather(tb, [rows, dcol])
                out = (c[0] + a * a, c[1] + a * b)
                if nacc == 3:
                    out = out + (c[2] + b * b,)
                return out

            return lax.fori_loop(0, DIM, body,
                                 (jnp.zeros((L,), jnp.float32),) * nacc)

        for sub in range(4):
            ee, eet = diag_dots(sub, 2)
            loc0[pl.ds(sub * L, L)] = ee
            loc1[pl.ds(sub * L, L)] = eet

        c1 = pltpu.async_copy(rel_emb.at[pl.ds(base_r, 4 * L)], ta, semP)
        c2 = pltpu.async_copy(rel_tr.at[pl.ds(base_r, 4 * L)], tb, semP)
        c1.wait()
        c2.wait()
        for sub in range(4):
            rr, rrt, qq = diag_dots(sub, 3)
            loc2[pl.ds(sub * L, L)] = rr
            loc3[pl.ds(sub * L, L)] = rrt
            loc4[pl.ds(sub * L, L)] = qq

        pltpu.sync_copy(loc0, sh0.at[pl.ds(base_e, 4 * L)])
        pltpu.sync_copy(loc1, sh1.at[pl.ds(base_e, 4 * L)])
        pltpu.sync_copy(loc2, sh2.at[pl.ds(base_r, 4 * L)])
        pltpu.sync_copy(loc3, sh3.at[pl.ds(base_r, 4 * L)])
        pltpu.sync_copy(loc4, sh4.at[pl.ds(base_r, 4 * L)])
        plsc.subcore_barrier()
        for shq, preq in ((sh0, pre0), (sh1, pre1), (sh2, pre2),
                          (sh3, pre3), (sh4, pre4)):
            pltpu.sync_copy(shq, preq)

        def distance(s, g, ho, to, ro):
            eb, wr, _ = s
            h_ids = eidx_v[g, pl.ds(ho * L, L)]
            t_ids = eidx_v[g, pl.ds(to * L, L)]
            r_ids = ridx_v[g, pl.ds(ro * L, L)]
            hh = plsc.load_gather(pre0, [h_ids])
            tt2 = plsc.load_gather(pre0, [t_ids])
            sh = plsc.load_gather(pre1, [h_ids])
            st = plsc.load_gather(pre1, [t_ids])
            rr = plsc.load_gather(pre2, [r_ids])
            rrt = plsc.load_gather(pre3, [r_ids])
            rtrt = plsc.load_gather(pre4, [r_ids])
            hr, ht_d, rt_d, hrt, trt = _cross_dots(
                eb, wr, lanes, ho * L, to * L, ro * L)
            nh2 = hh + 2.0 * sh * hrt + sh * sh * rtrt
            nt2 = tt2 + 2.0 * st * trt + st * st * rtrt
            hp_r = hr + sh * rrt
            hp_tp = ht_d + st * hrt + sh * trt + sh * st * rtrt
            r_tp = rt_d + st * rrt
            inh = _rsqrt(nh2)
            int_ = _rsqrt(nt2)
            inr = _rsqrt(rr)
            d2 = 3.0 + 2.0 * (hp_r * inh * inr - hp_tp * inh * int_
                              - r_tp * inr * int_)
            d2 = jnp.maximum(d2, 0.0)
            return d2 * _rsqrt(d2)

        def compute(s, g, acc):
            pos = distance(s, g, 0, 1, 0)
            neg = distance(s, g, 2, 3, 1)
            return acc + jnp.maximum(pos - neg + MARGIN, 0.0)

        def pair(gg, acc):
            drain(sets[0])
            acc = compute(sets[0], 2 * gg, acc)

            @pl.when(gg < GROUPS // 2 - 1)
            def _():
                fire(2 * gg + 2, sets[0])

            drain(sets[1])
            acc = compute(sets[1], 2 * gg + 1, acc)

            @pl.when(gg < GROUPS // 2 - 1)
            def _():
                fire(2 * gg + 3, sets[1])

            return acc

        acc = lax.fori_loop(0, GROUPS // 2, pair, jnp.zeros((L,), jnp.float32))
        acc_v[...] = acc
        pltpu.sync_copy(acc_v, out_hbm.at[wid])

    return sc_kernel


_SC_KERNEL = _make_sc_kernel()


def _pair_tables(a, b):
    au = lax.bitcast_convert_type(a, jnp.uint32) & jnp.uint32(0xFFFF0000)
    bu = lax.bitcast_convert_type(b, jnp.uint32) >> 16
    return lax.bitcast_convert_type(au | bu, jnp.int32)


@jax.jit
def kernel(current_triples, corrupted_triples, ent_embedding, rel_embedding,
           ent_transfer, rel_transfer):
    cur = current_triples.astype(jnp.int32)
    cor = corrupted_triples.astype(jnp.int32)

    def wg(col_arrays):
        parts = [a.reshape(NW, GROUPS, L) for a in col_arrays]
        return jnp.stack(parts, axis=2).reshape(NW, GROUPS, len(parts) * L)

    eidx = wg([cur[:, 0], cur[:, 2], cor[:, 0], cor[:, 2]])
    ridx = wg([cur[:, 1], cor[:, 1]])
    wrel = _pair_tables(rel_transfer, rel_embedding)
    partials = _SC_KERNEL(ent_embedding, ent_transfer, rel_embedding,
                          rel_transfer, wrel, eidx, ridx)
    return jnp.sum(partials) / BATCH
